# baseline (speedup 1.0000x reference)
import functools

import jax
import jax.numpy as jnp
from jax import lax
from jax.experimental import pallas as pl
from jax.experimental.pallas import tpu as pltpu
from jax.experimental.pallas import tpu_sc as plsc

EMB = 32
BATCH = 16384


def _make_kernel(vocab, emb, batch):
    info = plsc.get_sparse_core_info()
    nw = info.num_cores * info.num_subcores
    b_per_w = batch // nw
    mesh = plsc.VectorSubcoreMesh(core_axis_name="c", subcore_axis_name="s")

    @functools.partial(
        pl.kernel,
        mesh=mesh,
        out_type=[
            jax.ShapeDtypeStruct((batch, emb), jnp.float32),
            jax.ShapeDtypeStruct((batch, emb), jnp.float32),
        ],
        scratch_types=[
            pltpu.VMEM((b_per_w,), jnp.int32),
            pltpu.VMEM((b_per_w,), jnp.int32),
            pltpu.SemaphoreType.DMA,
        ],
    )
    def gather_kernel(table_hbm, uidx_hbm, iidx_hbm, out_u, out_i,
                      uvmem, ivmem, sem):
        wid = lax.axis_index("s") * info.num_cores + lax.axis_index("c")
        base = wid * b_per_w
        pltpu.sync_copy(uidx_hbm.at[pl.ds(base, b_per_w)], uvmem)
        pltpu.sync_copy(iidx_hbm.at[pl.ds(base, b_per_w)], ivmem)

        def fire(kb, _):
            k0 = kb * 16
            uv = uvmem[pl.ds(k0, 16)]
            iv = ivmem[pl.ds(k0, 16)]
            for j in range(16):
                pltpu.async_copy(table_hbm.at[pl.ds(uv[j], 1)],
                                 out_u.at[pl.ds(base + k0 + j, 1)], sem)
                pltpu.async_copy(table_hbm.at[pl.ds(iv[j], 1)],
                                 out_i.at[pl.ds(base + k0 + j, 1)], sem)
            return 0

        lax.fori_loop(0, b_per_w // 16, fire, 0)

        def drain(k, _):
            pltpu.make_async_copy(table_hbm.at[pl.ds(0, 1)],
                                  out_u.at[pl.ds(base, 1)], sem).wait()
            pltpu.make_async_copy(table_hbm.at[pl.ds(0, 1)],
                                  out_i.at[pl.ds(base, 1)], sem).wait()
            return 0

        lax.fori_loop(0, b_per_w, drain, 0)

    return gather_kernel


def kernel(embeddings, user_ids, item_ids):
    vocab, emb = embeddings.shape
    fn = _make_kernel(vocab, emb, user_ids.shape[0])
    users_emb, items_emb = fn(embeddings, user_ids, item_ids)
    return (users_emb, items_emb)


# speedup vs baseline: 1.0001x; 1.0001x over previous
import functools

import jax
import jax.numpy as jnp
from jax import lax
from jax.experimental import pallas as pl
from jax.experimental.pallas import tpu as pltpu
from jax.experimental.pallas import tpu_sc as plsc

EMB = 32
BATCH = 16384


def _make_kernel(vocab, emb, batch):
    info = plsc.get_sparse_core_info()
    nw = info.num_cores * info.num_subcores
    b_per_w = batch // nw
    mesh = plsc.VectorSubcoreMesh(core_axis_name="c", subcore_axis_name="s")

    @functools.partial(
        pl.kernel,
        mesh=mesh,
        out_type=[
            jax.ShapeDtypeStruct((batch, emb), jnp.float32),
            jax.ShapeDtypeStruct((batch, emb), jnp.float32),
        ],
        scratch_types=[
            pltpu.VMEM((b_per_w,), jnp.int32),
            pltpu.VMEM((b_per_w,), jnp.int32),
            pltpu.SemaphoreType.DMA,
        ],
    )
    def gather_kernel(table_hbm, uidx_hbm, iidx_hbm, out_u, out_i,
                      uvmem, ivmem, sem):
        wid = lax.axis_index("s") * info.num_cores + lax.axis_index("c")
        base = wid * b_per_w
        pltpu.sync_copy(uidx_hbm.at[pl.ds(base, b_per_w)], uvmem)
        pltpu.sync_copy(iidx_hbm.at[pl.ds(base, b_per_w)], ivmem)

        def fire(kb, _):
            k0 = kb * 16
            uv = uvmem[pl.ds(k0, 16)]
            iv = ivmem[pl.ds(k0, 16)]
            for j in range(16):
                pltpu.async_copy(table_hbm.at[pl.ds(uv[j], 1)],
                                 out_u.at[pl.ds(base + k0 + j, 1)], sem)
                pltpu.async_copy(table_hbm.at[pl.ds(iv[j], 1)],
                                 out_i.at[pl.ds(base + k0 + j, 1)], sem)
            return 0

        lax.fori_loop(0, b_per_w // 16, fire, 0)

        pltpu.make_async_copy(table_hbm.at[pl.ds(0, b_per_w)],
                              out_u.at[pl.ds(base, b_per_w)], sem).wait()
        pltpu.make_async_copy(table_hbm.at[pl.ds(0, b_per_w)],
                              out_i.at[pl.ds(base, b_per_w)], sem).wait()

    return gather_kernel


def kernel(embeddings, user_ids, item_ids):
    vocab, emb = embeddings.shape
    fn = _make_kernel(vocab, emb, user_ids.shape[0])
    users_emb, items_emb = fn(embeddings, user_ids, item_ids)
    return (users_emb, items_emb)


# speedup vs baseline: 4.0332x; 4.0329x over previous
import functools

import jax
import jax.numpy as jnp
from jax import lax
from jax.experimental import pallas as pl
from jax.experimental.pallas import tpu as pltpu
from jax.experimental.pallas import tpu_sc as plsc

EMB = 32
BATCH = 16384


def _make_kernel(ngroups, batch):
    info = plsc.get_sparse_core_info()
    nw = info.num_cores * info.num_subcores
    b_per_w = batch // nw
    mesh = plsc.VectorSubcoreMesh(core_axis_name="c", subcore_axis_name="s")

    @functools.partial(
        pl.kernel,
        mesh=mesh,
        out_type=[
            jax.ShapeDtypeStruct((batch // 8, 8, EMB), jnp.float32),
            jax.ShapeDtypeStruct((batch // 8, 8, EMB), jnp.float32),
        ],
        scratch_types=[
            pltpu.VMEM((2 * b_per_w,), jnp.int32),
            pltpu.VMEM((b_per_w // 8, 8, EMB), jnp.float32),
            pltpu.SemaphoreType.DMA,
        ],
        compiler_params=pltpu.CompilerParams(needs_layout_passes=False),
    )
    def gather_kernel(table_hbm, uidx_hbm, iidx_hbm, out_u, out_i,
                      idx_v, stage_v, sem):
        wid = lax.axis_index("s") * info.num_cores + lax.axis_index("c")
        base = wid * b_per_w
        pltpu.sync_copy(uidx_hbm.at[pl.ds(base, b_per_w)],
                        idx_v.at[pl.ds(0, b_per_w)])
        pltpu.sync_copy(iidx_hbm.at[pl.ds(base, b_per_w)],
                        idx_v.at[pl.ds(b_per_w, b_per_w)])

        def one_table(off, out):
            def fire(kb, _):
                k0 = off + kb * 16
                v = idx_v[pl.ds(k0, 16)]
                for j in range(16):
                    k = kb * 16 + j
                    pltpu.async_copy(
                        table_hbm.at[pl.ds(v[j] >> 3, 1), pl.ds(v[j] & 7, 1)],
                        stage_v.at[pl.ds(k // 8, 1), pl.ds(k % 8, 1)],
                        sem)
                return 0

            lax.fori_loop(0, b_per_w // 16, fire, 0)
            pltpu.make_async_copy(
                table_hbm.at[pl.ds(0, b_per_w // 8)], stage_v, sem).wait()
            pltpu.sync_copy(stage_v, out.at[pl.ds(base // 8, b_per_w // 8)])

        one_table(0, out_u)
        one_table(b_per_w, out_i)

    return gather_kernel


def kernel(embeddings, user_ids, item_ids):
    vocab, emb = embeddings.shape
    table3 = embeddings.reshape(vocab // 8, 8, emb)
    batch = user_ids.shape[0]
    fn = _make_kernel(vocab // 8, batch)
    users_emb, items_emb = fn(table3, user_ids, item_ids)
    return (users_emb.reshape(batch, emb), items_emb.reshape(batch, emb))


# speedup vs baseline: 4.0624x; 1.0072x over previous
import functools

import jax
import jax.numpy as jnp
from jax import lax
from jax.experimental import pallas as pl
from jax.experimental.pallas import tpu as pltpu
from jax.experimental.pallas import tpu_sc as plsc

EMB = 32
BATCH = 16384
CH = 128
NBUF = 4


def _make_kernel(ngroups, batch):
    info = plsc.get_sparse_core_info()
    nw = info.num_cores * info.num_subcores
    b_per_w = batch // nw
    nch = (2 * b_per_w) // CH
    mesh = plsc.VectorSubcoreMesh(core_axis_name="c", subcore_axis_name="s")

    @functools.partial(
        pl.kernel,
        mesh=mesh,
        out_type=[
            jax.ShapeDtypeStruct((batch // 8, 8, EMB), jnp.float32),
            jax.ShapeDtypeStruct((batch // 8, 8, EMB), jnp.float32),
        ],
        scratch_types=[
            pltpu.VMEM((2 * b_per_w,), jnp.int32),
        ]
        + [pltpu.VMEM((CH // 8, 8, EMB), jnp.float32) for _ in range(NBUF)]
        + [pltpu.SemaphoreType.DMA for _ in range(NBUF)],
        compiler_params=pltpu.CompilerParams(needs_layout_passes=False),
    )
    def gather_kernel(table_hbm, uidx_hbm, iidx_hbm, out_u, out_i,
                      idx_v, *bufs_sems):
        bufs = bufs_sems[:NBUF]
        sems = bufs_sems[NBUF:]
        wid = lax.axis_index("s") * info.num_cores + lax.axis_index("c")
        base = wid * b_per_w
        pltpu.sync_copy(uidx_hbm.at[pl.ds(base, b_per_w)],
                        idx_v.at[pl.ds(0, b_per_w)])
        pltpu.sync_copy(iidx_hbm.at[pl.ds(base, b_per_w)],
                        idx_v.at[pl.ds(b_per_w, b_per_w)])

        def fire(c, buf, sem):
            def blk(kb, _):
                v = idx_v[pl.ds(c * CH + kb * 16, 16)]
                for j in range(16):
                    pltpu.async_copy(
                        table_hbm.at[pl.ds(v[j] >> 3, 1), pl.ds(v[j] & 7, 1)],
                        buf.at[pl.ds(kb * 2 + j // 8, 1), pl.ds(j % 8, 1)],
                        sem)
                return 0

            lax.fori_loop(0, CH // 16, blk, 0)

        def wait_and_writeout(c, buf, sem):
            pltpu.make_async_copy(
                table_hbm.at[pl.ds(0, CH // 8)], buf, sem).wait()
            out = out_u if c < nch // 2 else out_i
            grp0 = (base + (c % (nch // 2)) * CH) // 8
            pltpu.sync_copy(buf, out.at[pl.ds(grp0, CH // 8)])

        for c in range(NBUF):
            fire(c, bufs[c], sems[c])
        for c in range(nch):
            p = c % NBUF
            wait_and_writeout(c, bufs[p], sems[p])
            if c + NBUF < nch:
                fire(c + NBUF, bufs[p], sems[p])

    return gather_kernel


def kernel(embeddings, user_ids, item_ids):
    vocab, emb = embeddings.shape
    table3 = embeddings.reshape(vocab // 8, 8, emb)
    batch = user_ids.shape[0]
    fn = _make_kernel(vocab // 8, batch)
    users_emb, items_emb = fn(table3, user_ids, item_ids)
    return (users_emb.reshape(batch, emb), items_emb.reshape(batch, emb))
